# Initial kernel scaffold; baseline (speedup 1.0000x reference)
import jax
import jax.numpy as jnp
from jax.experimental import pallas as pl


def kernel(tokens, wte_weight, learned_embedding):
    raise NotImplementedError("write your pallas kernel here")


# speedup vs baseline: 3.6661x; 3.6661x over previous
import jax
import jax.numpy as jnp
from jax import lax
from jax.experimental import pallas as pl
from jax.experimental.pallas import tpu as pltpu
from jax.experimental.pallas import tpu_sc as plsc

BATCH = 4096
SEQ = 200
EMBED_DIM = 64
N_TOKENS = 20

NUM_CORES = 2
NUM_SUBCORES = 16
NUM_WORKERS = NUM_CORES * NUM_SUBCORES
ROWS_PER_WORKER = BATCH // NUM_WORKERS

COL0 = 16
TOK_W = SEQ - COL0
SPLIT0 = 96
SPLIT1 = TOK_W - SPLIT0
G = 2
N_CHUNKS = ROWS_PER_WORKER // G


def _body(tokens_h, wte_h, learned_h, out_h,
          tok_v, stage0, stage1, gsem, osem0, osem1):
  wid = lax.axis_index("s") * NUM_CORES + lax.axis_index("c")
  base = wid * ROWS_PER_WORKER

  pltpu.sync_copy(
      tokens_h.at[pl.ds(base, ROWS_PER_WORKER), pl.ds(COL0, TOK_W)], tok_v)

  for st in (stage0, stage1):
    for b in range(G):
      pltpu.sync_copy(learned_h.at[pl.ds(0, COL0)], st.at[b, pl.ds(0, COL0)])

  def chunk(i, st, osem):
    copies = []
    for b in range(G):
      r = i * G + b
      copies.append(pltpu.async_copy(
          wte_h.at[tok_v.at[r, pl.ds(0, SPLIT0)]],
          st.at[b, pl.ds(COL0, SPLIT0)], gsem))
      copies.append(pltpu.async_copy(
          wte_h.at[tok_v.at[r, pl.ds(SPLIT0, SPLIT1)]],
          st.at[b, pl.ds(COL0 + SPLIT0, SPLIT1)], gsem))
    for cp in copies:
      cp.wait()
    patches = [
        pltpu.async_copy(
            learned_h.at[pl.ds(COL0, N_TOKENS - COL0)],
            st.at[b, pl.ds(COL0, N_TOKENS - COL0)], gsem)
        for b in range(G)
    ]
    for cp in patches:
      cp.wait()
    pltpu.async_copy(st, out_h.at[pl.ds(base + i * G, G)], osem)

  def loop_body(i0, carry):
    for p, (st, osem) in enumerate(((stage0, osem0), (stage1, osem1))):
      @pl.when(i0 > 0)
      def _wait():
        pltpu.make_async_copy(st, out_h.at[pl.ds(base, G)], osem).wait()
      chunk(i0 * 2 + p, st, osem)
    return carry

  lax.fori_loop(0, N_CHUNKS // 2, loop_body, 0)

  pltpu.make_async_copy(stage0, out_h.at[pl.ds(base, G)], osem0).wait()
  pltpu.make_async_copy(stage1, out_h.at[pl.ds(base, G)], osem1).wait()


@jax.jit
def _run(tokens, wte_weight, learned_embedding):
  mesh = plsc.VectorSubcoreMesh(
      core_axis_name="c", subcore_axis_name="s",
      num_cores=NUM_CORES, num_subcores=NUM_SUBCORES)
  return pl.kernel(
      _body,
      out_type=jax.ShapeDtypeStruct((BATCH, SEQ, EMBED_DIM), jnp.float32),
      mesh=mesh,
      compiler_params=pltpu.CompilerParams(use_tc_tiling_on_sc=False),
      scratch_types=[
          pltpu.VMEM((ROWS_PER_WORKER, TOK_W), jnp.int32),
          pltpu.VMEM((G, SEQ, EMBED_DIM), jnp.float32),
          pltpu.VMEM((G, SEQ, EMBED_DIM), jnp.float32),
          pltpu.SemaphoreType.DMA,
          pltpu.SemaphoreType.DMA,
          pltpu.SemaphoreType.DMA,
      ],
  )(tokens, wte_weight, learned_embedding)


def kernel(tokens, wte_weight, learned_embedding):
  return _run(tokens.astype(jnp.int32), wte_weight, learned_embedding)


# speedup vs baseline: 5.2887x; 1.4426x over previous
import jax
import jax.numpy as jnp
from jax import lax
from jax.experimental import pallas as pl
from jax.experimental.pallas import tpu as pltpu
from jax.experimental.pallas import tpu_sc as plsc

BATCH = 4096
SEQ = 200
EMBED_DIM = 64
N_TOKENS = 20
LANES = 16

NUM_CORES = 2
NUM_SUBCORES = 16
NUM_WORKERS = NUM_CORES * NUM_SUBCORES
ROWS_PER_WORKER = BATCH // NUM_WORKERS

COL0 = 16
TOK_W = SEQ - COL0
SPLIT0 = 96
SPLIT1 = TOK_W - SPLIT0
NBUF = 4
LOOKAHEAD = 2


def _body(tokens_h, wte_h, learned_h, out_h,
          tok_v, lv, st0, st1, st2, st3,
          g0, g1, g2, g3, o0, o1, o2, o3):
  stages = (st0, st1, st2, st3)
  gsems = (g0, g1, g2, g3)
  osems = (o0, o1, o2, o3)

  wid = lax.axis_index("s") * NUM_CORES + lax.axis_index("c")
  base = wid * ROWS_PER_WORKER

  pltpu.sync_copy(
      tokens_h.at[pl.ds(base, ROWS_PER_WORKER), pl.ds(COL0, TOK_W)], tok_v)
  pltpu.sync_copy(learned_h.at[pl.ds(COL0, N_TOKENS - COL0)], lv)
  patch = [[lv[k, pl.ds(c * LANES, LANES)] for c in range(EMBED_DIM // LANES)]
           for k in range(N_TOKENS - COL0)]

  for st in stages:
    pltpu.sync_copy(learned_h.at[pl.ds(0, COL0)], st.at[pl.ds(0, COL0)])

  def issue_gathers(r, st, gsem):
    pltpu.async_copy(
        wte_h.at[tok_v.at[r, pl.ds(0, SPLIT0)]],
        st.at[pl.ds(COL0, SPLIT0)], gsem)
    pltpu.async_copy(
        wte_h.at[tok_v.at[r, pl.ds(SPLIT0, SPLIT1)]],
        st.at[pl.ds(COL0 + SPLIT0, SPLIT1)], gsem)

  def drain_gathers(st, gsem):
    pltpu.make_async_copy(
        wte_h.at[pl.ds(0, SPLIT0)], st.at[pl.ds(COL0, SPLIT0)], gsem).wait()
    pltpu.make_async_copy(
        wte_h.at[pl.ds(0, SPLIT1)],
        st.at[pl.ds(COL0 + SPLIT0, SPLIT1)], gsem).wait()

  for r0 in range(LOOKAHEAD):
    issue_gathers(r0, stages[r0], gsems[r0])

  def loop_body(i0, carry):
    for p in range(NBUF):
      r = i0 * NBUF + p
      st, gsem, osem = stages[p], gsems[p], osems[p]
      drain_gathers(st, gsem)
      for k in range(N_TOKENS - COL0):
        for c in range(EMBED_DIM // LANES):
          st[COL0 + k, pl.ds(c * LANES, LANES)] = patch[k][c]
      pltpu.async_copy(st, out_h.at[base + r], osem)

      pn = (p + LOOKAHEAD) % NBUF
      stn, gsemn, osemn = stages[pn], gsems[pn], osems[pn]

      @pl.when(r + LOOKAHEAD < ROWS_PER_WORKER)
      def _ahead():
        @pl.when(r >= NBUF - LOOKAHEAD)
        def _reclaim():
          pltpu.make_async_copy(stn, out_h.at[base], osemn).wait()
        issue_gathers(r + LOOKAHEAD, stn, gsemn)
    return carry

  lax.fori_loop(0, ROWS_PER_WORKER // NBUF, loop_body, 0)

  for p in range(NBUF):
    pltpu.make_async_copy(stages[p], out_h.at[base], osems[p]).wait()


@jax.jit
def _run(tokens, wte_weight, learned_embedding):
  mesh = plsc.VectorSubcoreMesh(
      core_axis_name="c", subcore_axis_name="s",
      num_cores=NUM_CORES, num_subcores=NUM_SUBCORES)
  return pl.kernel(
      _body,
      out_type=jax.ShapeDtypeStruct((BATCH, SEQ, EMBED_DIM), jnp.float32),
      mesh=mesh,
      compiler_params=pltpu.CompilerParams(use_tc_tiling_on_sc=False),
      scratch_types=(
          [pltpu.VMEM((ROWS_PER_WORKER, TOK_W), jnp.int32),
           pltpu.VMEM((N_TOKENS - COL0, EMBED_DIM), jnp.float32)] +
          [pltpu.VMEM((SEQ, EMBED_DIM), jnp.float32)] * NBUF +
          [pltpu.SemaphoreType.DMA] * (2 * NBUF)
      ),
  )(tokens, wte_weight, learned_embedding)


def kernel(tokens, wte_weight, learned_embedding):
  return _run(tokens.astype(jnp.int32), wte_weight, learned_embedding)
